# baseline (speedup 1.0000x reference)
import jax
import jax.numpy as jnp
from jax import lax
from jax.experimental import pallas as pl
from jax.experimental.pallas import tpu as pltpu
from jax.experimental.pallas import tpu_sc as plsc

_N, _D, _E, _G, _DF = 2048, 768, 8, 2, 1536
_EG = _E // _G
_A = 2 * _N
_T3 = 256
_NT = _A // _T3
_NP = _NT + _E - 1
_NF = _E * _NT
_NW = 32
_BW = _A // _NW
_TW = _N // _NW


def _head_fold_body(w2e_ref, b2e_ref, wr_ref, v_ref, c_ref):
    wr = wr_ref[...]
    v_ref[...] = jnp.dot(w2e_ref[0], wr,
                         preferred_element_type=jnp.float32)[None]
    c_ref[...] = jnp.dot(b2e_ref[0], wr,
                         preferred_element_type=jnp.float32)[None]


def _run_head_fold(W2, b2, W_risk):
    return pl.pallas_call(
        _head_fold_body,
        grid=(_E,),
        in_specs=[
            pl.BlockSpec((1, _DF, _D), lambda e: (e, 0, 0)),
            pl.BlockSpec((1, 1, _D), lambda e: (e, 0, 0)),
            pl.BlockSpec((_D, 1), lambda e: (0, 0)),
        ],
        out_specs=[
            pl.BlockSpec((1, _DF, 1), lambda e: (e, 0, 0)),
            pl.BlockSpec((1, 1, 1), lambda e: (e, 0, 0)),
        ],
        out_shape=[
            jax.ShapeDtypeStruct((_E, _DF, 1), jnp.float32),
            jax.ShapeDtypeStruct((_E, 1, 1), jnp.float32),
        ],
    )(W2, b2[:, None, :], W_risk)


def _router_body(x_ref, wg_ref, we_ref,
                 pos_ref, g_ref, pt_ref, pe_ref, plo_ref, phi_ref):
    xt = x_ref[...]
    gl = jnp.dot(xt, wg_ref[...])
    el = jnp.dot(xt, we_ref[...])
    mg = jnp.max(gl, axis=1, keepdims=True)
    eg = jnp.exp(gl - mg)
    pg = eg / jnp.sum(eg, axis=1, keepdims=True)
    grp = []
    for gi in range(_G):
        sl = el[:, gi * _EG:(gi + 1) * _EG]
        mx = jnp.max(sl, axis=1, keepdims=True)
        ex = jnp.exp(sl - mx)
        grp.append(pg[:, gi:gi + 1] * (ex / jnp.sum(ex, axis=1,
                                                    keepdims=True)))
    probs = jnp.concatenate(grp, axis=1)

    iota = lax.broadcasted_iota(jnp.int32, (_N, _E), 1)
    v1 = jnp.max(probs, axis=1, keepdims=True)
    i1 = jnp.argmax(probs, axis=1)[:, None]
    sel1 = iota == i1
    masked = jnp.where(sel1, -jnp.inf, probs)
    v2 = jnp.max(masked, axis=1, keepdims=True)
    i2 = jnp.argmax(masked, axis=1)[:, None]
    sel2 = iota == i2
    m1f = sel1.astype(jnp.float32)
    m2f = sel2.astype(jnp.float32)
    den = v1 + v2 + 1e-9
    g_ref[:, 0:1] = v1 / den
    g_ref[:, 1:2] = v2 / den

    HP = lax.Precision.HIGHEST
    r128 = lax.broadcasted_iota(jnp.int32, (128, 128), 0)
    c128 = lax.broadcasted_iota(jnp.int32, (128, 128), 1)
    L128 = (c128 < r128).astype(jnp.float32)
    U128 = (r128 < c128).astype(jnp.float32)
    within = []
    bsums = []
    for mf in (m1f, m2f):
        for b in range(_N // 128):
            blk = mf[b * 128:(b + 1) * 128]
            within.append(lax.dot(L128, blk))
            bsums.append(jnp.sum(blk, axis=0, keepdims=True))
    BS = jnp.concatenate(bsums, axis=0)
    r32 = lax.broadcasted_iota(jnp.int32, (32, 32), 0)
    c32 = lax.broadcasted_iota(jnp.int32, (32, 32), 1)
    L32 = (c32 < r32).astype(jnp.float32)
    carry = lax.dot(L32, BS)
    tot = jnp.sum(BS, axis=0, keepdims=True)
    r8 = lax.broadcasted_iota(jnp.int32, (_E, _E), 0)
    c8 = lax.broadcasted_iota(jnp.int32, (_E, _E), 1)
    U8 = (r8 < c8).astype(jnp.float32)
    off = lax.dot(tot, U8, precision=HP)

    nb = _N // 128
    for s_i, mf in enumerate((m1f, m2f)):
        win = jnp.concatenate(within[s_i * nb:(s_i + 1) * nb], axis=0)
        car = carry[s_i * nb:(s_i + 1) * nb]
        car_tok = jnp.broadcast_to(car[:, None, :], (nb, 128, _E))
        car_tok = car_tok.reshape(_N, _E)
        field = off + car_tok + win
        pos_ref[:, s_i:s_i + 1] = jnp.sum(
            mf * field, axis=1, keepdims=True).astype(jnp.int32)

    ends = off + tot
    fcol = lax.broadcasted_iota(jnp.int32, (_NF, 1), 0)
    ecol = fcol // _NT
    tcol = (fcol % _NT).astype(jnp.float32)
    ecol8 = lax.broadcasted_iota(jnp.int32, (_NF, _E), 1)
    Scol = (jnp.broadcast_to(ecol, (_NF, _E)) == ecol8).astype(jnp.float32)
    off_col = lax.dot(Scol, jnp.transpose(off), precision=HP)
    end_col = lax.dot(Scol, jnp.transpose(ends), precision=HP)
    act_col = ((off_col < (tcol + 1.0) * _T3) &
               (end_col > tcol * _T3)).astype(jnp.float32)
    rank_col = lax.dot(L128, act_col)
    prow = lax.broadcasted_iota(jnp.int32, (_NF, _NP), 1).astype(jnp.float32)
    PmatT = (jnp.broadcast_to(rank_col, (_NF, _NP)) == prow)
    PmatT = PmatT.astype(jnp.float32) * jnp.broadcast_to(act_col, (_NF, _NP))
    frow = lax.broadcasted_iota(jnp.int32, (1, _NF), 1)
    trow = (frow % _NT).astype(jnp.float32)
    erow = (frow // _NT).astype(jnp.float32)
    lo_row = jnp.transpose(off_col)
    hi_row = jnp.transpose(end_col)
    pt_ref[...] = lax.dot(trow, PmatT, precision=HP).astype(jnp.int32)
    pe_ref[...] = lax.dot(erow, PmatT, precision=HP).astype(jnp.int32)
    plo_ref[...] = lax.dot(lo_row, PmatT, precision=HP).astype(jnp.int32)
    phi_ref[...] = lax.dot(hi_row, PmatT, precision=HP).astype(jnp.int32)


def _run_router(x, Wg_group, Wg_expert):
    return pl.pallas_call(
        _router_body,
        grid=(1,),
        in_specs=[
            pl.BlockSpec((_N, _D), lambda i: (0, 0)),
            pl.BlockSpec((_D, _G), lambda i: (0, 0)),
            pl.BlockSpec((_D, _E), lambda i: (0, 0)),
        ],
        out_specs=[
            pl.BlockSpec((_N, 2), lambda i: (0, 0)),
            pl.BlockSpec((_N, 2), lambda i: (0, 0)),
            pl.BlockSpec((1, _NP), lambda i: (0, 0)),
            pl.BlockSpec((1, _NP), lambda i: (0, 0)),
            pl.BlockSpec((1, _NP), lambda i: (0, 0)),
            pl.BlockSpec((1, _NP), lambda i: (0, 0)),
        ],
        out_shape=[
            jax.ShapeDtypeStruct((_N, 2), jnp.int32),
            jax.ShapeDtypeStruct((_N, 2), jnp.float32),
            jax.ShapeDtypeStruct((1, _NP), jnp.int32),
            jax.ShapeDtypeStruct((1, _NP), jnp.int32),
            jax.ShapeDtypeStruct((1, _NP), jnp.int32),
            jax.ShapeDtypeStruct((1, _NP), jnp.int32),
        ],
    )(x, Wg_group, Wg_expert)


_BH = _BW // 2


def _dispatch_body(x_hbm, pos_hbm, xs_hbm,
                   pv2_v, posa_v, posb_v, rowsa_v, rowsb_v, semr, semw):
    wid = lax.axis_index("s") * 2 + lax.axis_index("c")
    slot = wid // (_NW // 2)
    btok = (wid % (_NW // 2)) * _BW
    pltpu.sync_copy(pos_hbm.at[pl.ds(btok, _BW)], pv2_v)
    ra = pltpu.async_copy(x_hbm.at[pl.ds(btok, _BH)], rowsa_v, semr)
    rb = pltpu.async_copy(x_hbm.at[pl.ds(btok + _BH, _BH)], rowsb_v, semr)
    scol = lax.iota(jnp.int32, 16) * 0 + slot
    for j in range(_BW // 16):
        ridx = lax.iota(jnp.int32, 16) + j * 16
        vals = plsc.load_gather(pv2_v, [ridx, scol])
        if j < _BH // 16:
            posa_v[pl.ds(j * 16, 16)] = vals
        else:
            posb_v[pl.ds((j - _BH // 16) * 16, 16)] = vals
    ra.wait()
    wa = pltpu.async_copy(rowsa_v, xs_hbm.at[posa_v], semw)
    rb.wait()
    wb = pltpu.async_copy(rowsb_v, xs_hbm.at[posb_v], semw)
    wa.wait()
    wb.wait()


def _run_dispatch(x, pos2):
    mesh = plsc.VectorSubcoreMesh(core_axis_name="c", subcore_axis_name="s")
    return pl.kernel(
        _dispatch_body,
        out_type=jax.ShapeDtypeStruct((_A, _D), jnp.float32),
        mesh=mesh,
        compiler_params=pltpu.CompilerParams(needs_layout_passes=False),
        scratch_types=[
            pltpu.VMEM((_BW, 2), jnp.int32),
            pltpu.VMEM((_BH,), jnp.int32),
            pltpu.VMEM((_BH,), jnp.int32),
            pltpu.VMEM((_BH, _D), jnp.float32),
            pltpu.VMEM((_BH, _D), jnp.float32),
            pltpu.SemaphoreType.DMA,
            pltpu.SemaphoreType.DMA,
        ],
    )(x, pos2)


def _ffn_body(tile_ref, exp_ref, lo_ref, hi_ref,
              xs_ref, w1_ref, b1_ref, w2_ref, b2_ref, wr_ref, out_ref,
              v_s, c_s):
    p = pl.program_id(0)
    t = tile_ref[0, p]
    pm = jnp.maximum(p - 1, 0)
    changed = (p == 0) | (exp_ref[0, p] != exp_ref[0, pm])

    @pl.when(changed)
    def _():
        wr = wr_ref[...]
        v_s[...] = jnp.dot(w2_ref[0], wr, preferred_element_type=jnp.float32)
        c_s[...] = jnp.dot(b2_ref[0], wr, preferred_element_type=jnp.float32)

    xs = xs_ref[...]
    h = jnp.dot(xs.astype(jnp.bfloat16), w1_ref[0].astype(jnp.bfloat16),
                preferred_element_type=jnp.float32) + b1_ref[0, 0][None]
    h = jax.nn.gelu(h)
    s = jnp.dot(h, v_s[...])
    val = s + c_s[0, 0]
    rows = t * _T3 + lax.broadcasted_iota(jnp.int32, (_T3, 1), 0)
    m = (rows >= lo_ref[0, p]) & (rows < hi_ref[0, p])
    val = jnp.where(m, val, 0.0)

    @pl.when(p == 0)
    def _():
        out_ref[...] = jnp.zeros((_A, 1), jnp.float32)

    out_ref[pl.ds(t * _T3, _T3), :] = out_ref[pl.ds(t * _T3, _T3), :] + val


def _run_ffn(x_sorted, W1, b1, W2, b2, W_risk, passes):
    p_tile, p_exp, p_lo, p_hi = passes
    grid_spec = pltpu.PrefetchScalarGridSpec(
        num_scalar_prefetch=4,
        grid=(_NP,),
        in_specs=[
            pl.BlockSpec((_T3, _D), lambda p, tr, er, lr, hr: (tr[0, p], 0)),
            pl.BlockSpec((1, _D, _DF),
                         lambda p, tr, er, lr, hr: (er[0, p], 0, 0)),
            pl.BlockSpec((1, 1, _DF),
                         lambda p, tr, er, lr, hr: (er[0, p], 0, 0)),
            pl.BlockSpec((1, _DF, _D),
                         lambda p, tr, er, lr, hr: (er[0, p], 0, 0)),
            pl.BlockSpec((1, 1, _D),
                         lambda p, tr, er, lr, hr: (er[0, p], 0, 0)),
            pl.BlockSpec((_D, 1), lambda p, tr, er, lr, hr: (0, 0)),
        ],
        out_specs=pl.BlockSpec((_A, 1), lambda p, tr, er, lr, hr: (0, 0)),
        scratch_shapes=[
            pltpu.VMEM((_DF, 1), jnp.float32),
            pltpu.VMEM((1, 1), jnp.float32),
        ],
    )
    return pl.pallas_call(
        _ffn_body,
        grid_spec=grid_spec,
        out_shape=jax.ShapeDtypeStruct((_A, 1), jnp.float32),
    )(p_tile, p_exp, p_lo, p_hi, x_sorted, W1, b1[:, None, :],
      W2, b2[:, None, :], W_risk)


def _combine_body(sp_hbm, pos_hbm, g_hbm, br_hbm, out_hbm,
                  sp_v, pv, gv, br_v, r_v):
    wid = lax.axis_index("s") * 2 + lax.axis_index("c")
    base = wid * _TW
    pltpu.sync_copy(sp_hbm, sp_v)
    pltpu.sync_copy(pos_hbm.at[pl.ds(base, _TW)], pv)
    pltpu.sync_copy(g_hbm.at[pl.ds(base, _TW)], gv)
    pltpu.sync_copy(br_hbm, br_v)
    brv = br_v[...]
    zc = lax.iota(jnp.int32, 16) * 0
    oc = zc + 1
    for i in range(_TW // 16):
        ridx = lax.iota(jnp.int32, 16) + i * 16
        p0 = plsc.load_gather(pv, [ridx, zc])
        p1 = plsc.load_gather(pv, [ridx, oc])
        g0 = plsc.load_gather(gv, [ridx, zc])
        g1 = plsc.load_gather(gv, [ridx, oc])
        a0 = plsc.load_gather(sp_v, [p0])
        a1 = plsc.load_gather(sp_v, [p1])
        r_v[pl.ds(i * 16, 16)] = g0 * a0 + g1 * a1 + brv
    pltpu.sync_copy(r_v, out_hbm.at[pl.ds(base, _TW)])


def _run_combine(s_flat, pos2, g2, br16):
    mesh = plsc.VectorSubcoreMesh(core_axis_name="c", subcore_axis_name="s")
    return pl.kernel(
        _combine_body,
        out_type=jax.ShapeDtypeStruct((_N,), jnp.float32),
        mesh=mesh,
        compiler_params=pltpu.CompilerParams(needs_layout_passes=False),
        scratch_types=[
            pltpu.VMEM((_A,), jnp.float32),
            pltpu.VMEM((_TW, 2), jnp.int32),
            pltpu.VMEM((_TW, 2), jnp.float32),
            pltpu.VMEM((16,), jnp.float32),
            pltpu.VMEM((_TW,), jnp.float32),
        ],
    )(s_flat, pos2, g2, br16)


@jax.jit
def kernel(x, Wg_group, Wg_expert, W1, b1, W2, b2, W_risk, b_risk):
    pos2, g2, p_tile, p_exp, p_lo, p_hi = _run_router(x, Wg_group, Wg_expert)
    x_sorted = _run_dispatch(x, pos2)
    s = _run_ffn(x_sorted, W1, b1, W2, b2, W_risk,
                 (p_tile, p_exp, p_lo, p_hi))
    br16 = jnp.broadcast_to(b_risk, (16,)).astype(jnp.float32)
    return _run_combine(s.reshape(_A), pos2, g2, br16)


# speedup vs baseline: 1.0398x; 1.0398x over previous
import jax
import jax.numpy as jnp
from jax import lax
from jax.experimental import pallas as pl
from jax.experimental.pallas import tpu as pltpu
from jax.experimental.pallas import tpu_sc as plsc

_N, _D, _E, _G, _DF = 2048, 768, 8, 2, 1536
_EG = _E // _G
_A = 2 * _N
_T3 = 256
_NT = _A // _T3
_NP = _NT + _E - 1
_NF = _E * _NT
_NW = 32
_BW = _A // _NW
_TW = _N // _NW


def _head_fold_body(w2e_ref, b2e_ref, wr_ref, v_ref, c_ref):
    wr = wr_ref[...]
    v_ref[...] = jnp.dot(w2e_ref[0], wr,
                         preferred_element_type=jnp.float32)[None]
    c_ref[...] = jnp.dot(b2e_ref[0], wr,
                         preferred_element_type=jnp.float32)[None]


def _run_head_fold(W2, b2, W_risk):
    return pl.pallas_call(
        _head_fold_body,
        grid=(_E,),
        in_specs=[
            pl.BlockSpec((1, _DF, _D), lambda e: (e, 0, 0)),
            pl.BlockSpec((1, 1, _D), lambda e: (e, 0, 0)),
            pl.BlockSpec((_D, 1), lambda e: (0, 0)),
        ],
        out_specs=[
            pl.BlockSpec((1, _DF, 1), lambda e: (e, 0, 0)),
            pl.BlockSpec((1, 1, 1), lambda e: (e, 0, 0)),
        ],
        out_shape=[
            jax.ShapeDtypeStruct((_E, _DF, 1), jnp.float32),
            jax.ShapeDtypeStruct((_E, 1, 1), jnp.float32),
        ],
    )(W2, b2[:, None, :], W_risk)


def _router_body(x_ref, wg_ref, we_ref,
                 pos_ref, g_ref, pt_ref, pe_ref, plo_ref, phi_ref):
    xt = x_ref[...]
    gl = jnp.dot(xt, wg_ref[...])
    el = jnp.dot(xt, we_ref[...])
    mg = jnp.max(gl, axis=1, keepdims=True)
    eg = jnp.exp(gl - mg)
    pg = eg / jnp.sum(eg, axis=1, keepdims=True)
    grp = []
    for gi in range(_G):
        sl = el[:, gi * _EG:(gi + 1) * _EG]
        mx = jnp.max(sl, axis=1, keepdims=True)
        ex = jnp.exp(sl - mx)
        grp.append(pg[:, gi:gi + 1] * (ex / jnp.sum(ex, axis=1,
                                                    keepdims=True)))
    probs = jnp.concatenate(grp, axis=1)

    iota = lax.broadcasted_iota(jnp.int32, (_N, _E), 1)
    v1 = jnp.max(probs, axis=1, keepdims=True)
    i1 = jnp.argmax(probs, axis=1)[:, None]
    sel1 = iota == i1
    masked = jnp.where(sel1, -jnp.inf, probs)
    v2 = jnp.max(masked, axis=1, keepdims=True)
    i2 = jnp.argmax(masked, axis=1)[:, None]
    sel2 = iota == i2
    m1f = sel1.astype(jnp.float32)
    m2f = sel2.astype(jnp.float32)
    den = v1 + v2 + 1e-9
    g_ref[:, 0:1] = v1 / den
    g_ref[:, 1:2] = v2 / den

    HP = lax.Precision.HIGHEST
    r128 = lax.broadcasted_iota(jnp.int32, (128, 128), 0)
    c128 = lax.broadcasted_iota(jnp.int32, (128, 128), 1)
    L128 = (c128 < r128).astype(jnp.float32)
    U128 = (r128 < c128).astype(jnp.float32)
    within = []
    bsums = []
    for mf in (m1f, m2f):
        for b in range(_N // 128):
            blk = mf[b * 128:(b + 1) * 128]
            within.append(lax.dot(L128, blk))
            bsums.append(jnp.sum(blk, axis=0, keepdims=True))
    BS = jnp.concatenate(bsums, axis=0)
    r32 = lax.broadcasted_iota(jnp.int32, (32, 32), 0)
    c32 = lax.broadcasted_iota(jnp.int32, (32, 32), 1)
    L32 = (c32 < r32).astype(jnp.float32)
    carry = lax.dot(L32, BS)
    tot = jnp.sum(BS, axis=0, keepdims=True)
    r8 = lax.broadcasted_iota(jnp.int32, (_E, _E), 0)
    c8 = lax.broadcasted_iota(jnp.int32, (_E, _E), 1)
    U8 = (r8 < c8).astype(jnp.float32)
    off = lax.dot(tot, U8, precision=HP)

    nb = _N // 128
    for s_i, mf in enumerate((m1f, m2f)):
        win = jnp.concatenate(within[s_i * nb:(s_i + 1) * nb], axis=0)
        car = carry[s_i * nb:(s_i + 1) * nb]
        car_tok = jnp.broadcast_to(car[:, None, :], (nb, 128, _E))
        car_tok = car_tok.reshape(_N, _E)
        field = off + car_tok + win
        pos_ref[:, s_i:s_i + 1] = jnp.sum(
            mf * field, axis=1, keepdims=True).astype(jnp.int32)

    ends = off + tot
    fcol = lax.broadcasted_iota(jnp.int32, (_NF, 1), 0)
    ecol = fcol // _NT
    tcol = (fcol % _NT).astype(jnp.float32)
    ecol8 = lax.broadcasted_iota(jnp.int32, (_NF, _E), 1)
    Scol = (jnp.broadcast_to(ecol, (_NF, _E)) == ecol8).astype(jnp.float32)
    off_col = lax.dot(Scol, jnp.transpose(off), precision=HP)
    end_col = lax.dot(Scol, jnp.transpose(ends), precision=HP)
    act_col = ((off_col < (tcol + 1.0) * _T3) &
               (end_col > tcol * _T3)).astype(jnp.float32)
    rank_col = lax.dot(L128, act_col)
    prow = lax.broadcasted_iota(jnp.int32, (_NF, _NP), 1).astype(jnp.float32)
    PmatT = (jnp.broadcast_to(rank_col, (_NF, _NP)) == prow)
    PmatT = PmatT.astype(jnp.float32) * jnp.broadcast_to(act_col, (_NF, _NP))
    frow = lax.broadcasted_iota(jnp.int32, (1, _NF), 1)
    trow = (frow % _NT).astype(jnp.float32)
    erow = (frow // _NT).astype(jnp.float32)
    lo_row = jnp.transpose(off_col)
    hi_row = jnp.transpose(end_col)
    pt_ref[...] = lax.dot(trow, PmatT, precision=HP).astype(jnp.int32)
    pe_ref[...] = lax.dot(erow, PmatT, precision=HP).astype(jnp.int32)
    plo_ref[...] = lax.dot(lo_row, PmatT, precision=HP).astype(jnp.int32)
    phi_ref[...] = lax.dot(hi_row, PmatT, precision=HP).astype(jnp.int32)


def _run_router(x, Wg_group, Wg_expert):
    return pl.pallas_call(
        _router_body,
        grid=(1,),
        in_specs=[
            pl.BlockSpec((_N, _D), lambda i: (0, 0)),
            pl.BlockSpec((_D, _G), lambda i: (0, 0)),
            pl.BlockSpec((_D, _E), lambda i: (0, 0)),
        ],
        out_specs=[
            pl.BlockSpec((_N, 2), lambda i: (0, 0)),
            pl.BlockSpec((_N, 2), lambda i: (0, 0)),
            pl.BlockSpec((1, _NP), lambda i: (0, 0)),
            pl.BlockSpec((1, _NP), lambda i: (0, 0)),
            pl.BlockSpec((1, _NP), lambda i: (0, 0)),
            pl.BlockSpec((1, _NP), lambda i: (0, 0)),
        ],
        out_shape=[
            jax.ShapeDtypeStruct((_N, 2), jnp.int32),
            jax.ShapeDtypeStruct((_N, 2), jnp.float32),
            jax.ShapeDtypeStruct((1, _NP), jnp.int32),
            jax.ShapeDtypeStruct((1, _NP), jnp.int32),
            jax.ShapeDtypeStruct((1, _NP), jnp.int32),
            jax.ShapeDtypeStruct((1, _NP), jnp.int32),
        ],
    )(x, Wg_group, Wg_expert)


_BH = _BW // 2


def _dispatch_body(x_hbm, pos_hbm, xs_hbm,
                   pv2_v, posa_v, posb_v, rowsa_v, rowsb_v, semr, semw):
    wid = lax.axis_index("s") * 2 + lax.axis_index("c")
    slot = wid // (_NW // 2)
    btok = (wid % (_NW // 2)) * _BW
    pltpu.sync_copy(pos_hbm.at[pl.ds(btok, _BW)], pv2_v)
    ra = pltpu.async_copy(x_hbm.at[pl.ds(btok, _BH)], rowsa_v, semr)
    rb = pltpu.async_copy(x_hbm.at[pl.ds(btok + _BH, _BH)], rowsb_v, semr)
    scol = lax.iota(jnp.int32, 16) * 0 + slot
    for j in range(_BW // 16):
        ridx = lax.iota(jnp.int32, 16) + j * 16
        vals = plsc.load_gather(pv2_v, [ridx, scol])
        if j < _BH // 16:
            posa_v[pl.ds(j * 16, 16)] = vals
        else:
            posb_v[pl.ds((j - _BH // 16) * 16, 16)] = vals
    ra.wait()
    wa = pltpu.async_copy(rowsa_v, xs_hbm.at[posa_v], semw)
    rb.wait()
    wb = pltpu.async_copy(rowsb_v, xs_hbm.at[posb_v], semw)
    wa.wait()
    wb.wait()


def _run_dispatch(x, pos2):
    mesh = plsc.VectorSubcoreMesh(core_axis_name="c", subcore_axis_name="s")
    return pl.kernel(
        _dispatch_body,
        out_type=jax.ShapeDtypeStruct((_A, _D), jnp.float32),
        mesh=mesh,
        compiler_params=pltpu.CompilerParams(needs_layout_passes=False),
        scratch_types=[
            pltpu.VMEM((_BW, 2), jnp.int32),
            pltpu.VMEM((_BH,), jnp.int32),
            pltpu.VMEM((_BH,), jnp.int32),
            pltpu.VMEM((_BH, _D), jnp.float32),
            pltpu.VMEM((_BH, _D), jnp.float32),
            pltpu.SemaphoreType.DMA,
            pltpu.SemaphoreType.DMA,
        ],
    )(x, pos2)


def _ffn_body(tile_ref, exp_ref, lo_ref, hi_ref,
              xs_ref, w1_ref, b1_ref, v_ref, c_ref, out_ref):
    p = pl.program_id(0)
    t = tile_ref[0, p]
    xs = xs_ref[...]
    h = jnp.dot(xs.astype(jnp.bfloat16), w1_ref[0].astype(jnp.bfloat16),
                preferred_element_type=jnp.float32) + b1_ref[0, 0][None]
    h = jax.nn.gelu(h)
    s = jnp.dot(h, v_ref[0])
    val = s + c_ref[0, 0, 0]
    rows = t * _T3 + lax.broadcasted_iota(jnp.int32, (_T3, 1), 0)
    m = (rows >= lo_ref[0, p]) & (rows < hi_ref[0, p])
    val = jnp.where(m, val, 0.0)

    @pl.when(p == 0)
    def _():
        out_ref[...] = jnp.zeros((_A, 1), jnp.float32)

    out_ref[pl.ds(t * _T3, _T3), :] = out_ref[pl.ds(t * _T3, _T3), :] + val


def _run_ffn(x_sorted, W1, b1, v, c, passes):
    p_tile, p_exp, p_lo, p_hi = passes
    grid_spec = pltpu.PrefetchScalarGridSpec(
        num_scalar_prefetch=4,
        grid=(_NP,),
        in_specs=[
            pl.BlockSpec((_T3, _D), lambda p, tr, er, lr, hr: (tr[0, p], 0)),
            pl.BlockSpec((1, _D, _DF),
                         lambda p, tr, er, lr, hr: (er[0, p], 0, 0)),
            pl.BlockSpec((1, 1, _DF),
                         lambda p, tr, er, lr, hr: (er[0, p], 0, 0)),
            pl.BlockSpec((1, _DF, 1),
                         lambda p, tr, er, lr, hr: (er[0, p], 0, 0)),
            pl.BlockSpec((1, 1, 1),
                         lambda p, tr, er, lr, hr: (er[0, p], 0, 0)),
        ],
        out_specs=pl.BlockSpec((_A, 1), lambda p, tr, er, lr, hr: (0, 0)),
    )
    return pl.pallas_call(
        _ffn_body,
        grid_spec=grid_spec,
        out_shape=jax.ShapeDtypeStruct((_A, 1), jnp.float32),
    )(p_tile, p_exp, p_lo, p_hi, x_sorted, W1, b1[:, None, :], v, c)


def _combine_body(sp_hbm, pos_hbm, g_hbm, br_hbm, out_hbm,
                  sp_v, pv, gv, br_v, r_v):
    wid = lax.axis_index("s") * 2 + lax.axis_index("c")
    base = wid * _TW
    pltpu.sync_copy(sp_hbm, sp_v)
    pltpu.sync_copy(pos_hbm.at[pl.ds(base, _TW)], pv)
    pltpu.sync_copy(g_hbm.at[pl.ds(base, _TW)], gv)
    pltpu.sync_copy(br_hbm, br_v)
    brv = br_v[...]
    zc = lax.iota(jnp.int32, 16) * 0
    oc = zc + 1
    for i in range(_TW // 16):
        ridx = lax.iota(jnp.int32, 16) + i * 16
        p0 = plsc.load_gather(pv, [ridx, zc])
        p1 = plsc.load_gather(pv, [ridx, oc])
        g0 = plsc.load_gather(gv, [ridx, zc])
        g1 = plsc.load_gather(gv, [ridx, oc])
        a0 = plsc.load_gather(sp_v, [p0])
        a1 = plsc.load_gather(sp_v, [p1])
        r_v[pl.ds(i * 16, 16)] = g0 * a0 + g1 * a1 + brv
    pltpu.sync_copy(r_v, out_hbm.at[pl.ds(base, _TW)])


def _run_combine(s_flat, pos2, g2, br16):
    mesh = plsc.VectorSubcoreMesh(core_axis_name="c", subcore_axis_name="s")
    return pl.kernel(
        _combine_body,
        out_type=jax.ShapeDtypeStruct((_N,), jnp.float32),
        mesh=mesh,
        compiler_params=pltpu.CompilerParams(needs_layout_passes=False),
        scratch_types=[
            pltpu.VMEM((_A,), jnp.float32),
            pltpu.VMEM((_TW, 2), jnp.int32),
            pltpu.VMEM((_TW, 2), jnp.float32),
            pltpu.VMEM((16,), jnp.float32),
            pltpu.VMEM((_TW,), jnp.float32),
        ],
    )(s_flat, pos2, g2, br16)


@jax.jit
def kernel(x, Wg_group, Wg_expert, W1, b1, W2, b2, W_risk, b_risk):
    pos2, g2, p_tile, p_exp, p_lo, p_hi = _run_router(x, Wg_group, Wg_expert)
    v, c = _run_head_fold(W2, b2, W_risk)
    x_sorted = _run_dispatch(x, pos2)
    s = _run_ffn(x_sorted, W1, b1, v, c, (p_tile, p_exp, p_lo, p_hi))
    br16 = jnp.broadcast_to(b_risk, (16,)).astype(jnp.float32)
    return _run_combine(s.reshape(_A), pos2, g2, br16)
